# baseline (speedup 1.0000x reference)
import dataclasses
import functools

import jax
import jax.numpy as jnp
from jax import lax
from jax.experimental import pallas as pl
from jax.experimental.pallas import tpu as pltpu
from jax.experimental.pallas import tpu_sc as plsc

N = 320000
D = 128
K = 1000
KP = 1024
NC = 2
NS = 16
NW = NC * NS
CHUNK = N // NW
BLK = 80
NB = 5
NBLK = CHUNK // BLK
NOUT = CHUNK // (BLK * NB)
ZR = KP // NS


def _sc_segment_sums(embeddings, labels):
  mesh = plsc.VectorSubcoreMesh(core_axis_name="c", subcore_axis_name="s")
  cparams = dataclasses.replace(pltpu.CompilerParams(),
                                needs_layout_passes=False)

  @functools.partial(
      pl.kernel,
      out_type=[
          jax.ShapeDtypeStruct((NC, KP, D), jnp.float32),
          jax.ShapeDtypeStruct((NC, KP), jnp.float32),
      ],
      mesh=mesh,
      compiler_params=cparams,
      scratch_types=[
              pltpu.VMEM((KP,), jnp.float32),
              pltpu.VMEM((ZR, D), jnp.float32),
              pltpu.VMEM((NS, ZR), jnp.float32),
              pltpu.VMEM((ZR,), jnp.float32),
              pltpu.VMEM_SHARED((KP, D), jnp.float32),
              pltpu.VMEM_SHARED((NS, KP), jnp.float32),
      ],
  )
  def kern(emb_hbm, lab_hbm, sums_hbm, cnts_hbm, cnt_v,
           zrow_v, red_v, cout_v, ssums, scnt_s, *sems):
    ci = lax.axis_index("c")
    si = lax.axis_index("s")
    wid = ci * NS + si
    base = wid * CHUNK

    zero16 = jnp.zeros((16,), jnp.float32)
    one16 = jnp.full((16,), 1.0, jnp.float32)

    @pl.loop(0, ZR)
    def _(r):
      @pl.loop(0, D, step=16)
      def _(cc):
        zrow_v[r, pl.ds(cc, 16)] = zero16

    @pl.loop(0, KP, step=16)
    def _(r):
      cnt_v[pl.ds(r, 16)] = zero16

    pltpu.sync_copy(zrow_v, ssums.at[pl.ds(si * ZR, ZR)])
    plsc.subcore_barrier()

    def body(rows_vm, lab_vm):
      pltpu.sync_copy(rows_vm, ssums.at[lab_vm.at[0]], add=True)
      for g in range(BLK // 16):
        idx = lab_vm[0, pl.ds(g * 16, 16)]
        plsc.addupdate_scatter(cnt_v, [idx], one16)

    pltpu.emit_pipeline(
        body,
        grid=(N // BLK,),
        in_specs=[
            pl.BlockSpec((BLK, D), lambda i: (i, 0)),
            pl.BlockSpec((1, BLK), lambda i: (i, 0)),
        ],
        core_axis_name=("c", "s"),
        dimension_semantics=(pltpu.PARALLEL,),
    )(emb_hbm, lab_hbm)

    pltpu.sync_copy(cnt_v, scnt_s.at[si])
    plsc.subcore_barrier()
    for r in range(NS):
      pltpu.sync_copy(scnt_s.at[r, pl.ds(si * ZR, ZR)], red_v.at[r])
    for c in range(0, ZR, 16):
      acc = zero16
      for r in range(NS):
        acc = acc + red_v[r, pl.ds(c, 16)]
      cout_v[pl.ds(c, 16)] = acc
    pltpu.sync_copy(cout_v, cnts_hbm.at[ci, pl.ds(si * ZR, ZR)])

    pltpu.sync_copy(ssums.at[pl.ds(si * ZR, ZR)],
                    sums_hbm.at[ci, pl.ds(si * ZR, ZR)])

  return kern(embeddings, labels)


def _tc_finalize(psums, pcnts):
  def body(s_ref, c_ref, o_ref):
    s = s_ref[0] + s_ref[1]
    cnt = c_ref[0] + c_ref[1]
    pm = (cnt > 0.0).astype(jnp.float32)
    npres = jnp.sum(pm)
    means = s / jnp.maximum(cnt, 1.0)
    overall = jnp.sum(means * pm, axis=0, keepdims=True) / npres
    diff = (means - overall) * pm
    var = jnp.sum(diff * diff, axis=0, keepdims=True) / (npres - 1.0)
    o_ref[...] = jnp.broadcast_to(-jnp.mean(var), (1, 1))

  return pl.pallas_call(
      body,
      out_shape=jax.ShapeDtypeStruct((1, 1), jnp.float32),
  )(psums, pcnts)


def kernel(embeddings, labels):
  labels = labels.astype(jnp.int32).reshape(N // BLK, BLK)
  psums, pcnts = _sc_segment_sums(embeddings, labels)
  return _tc_finalize(psums, pcnts.reshape(NC, KP, 1))[0, 0]


# speedup vs baseline: 1.0058x; 1.0058x over previous
import dataclasses
import functools

import jax
import jax.numpy as jnp
from jax import lax
from jax.experimental import pallas as pl
from jax.experimental.pallas import tpu as pltpu
from jax.experimental.pallas import tpu_sc as plsc

N = 320000
D = 128
K = 1000
KP = 1024
NC = 2
NS = 16
NW = NC * NS
CHUNK = N // NW
BLK = 80
NB = 3
SUB = 3
LBLK = BLK * SUB
NFULL = 41
TSUB = 2
NBLK = CHUNK // BLK
ZR = KP // NS


def _sc_segment_sums(embeddings, labels):
  mesh = plsc.VectorSubcoreMesh(core_axis_name="c", subcore_axis_name="s")
  cparams = dataclasses.replace(pltpu.CompilerParams(),
                                needs_layout_passes=False)

  @functools.partial(
      pl.kernel,
      out_type=[
          jax.ShapeDtypeStruct((NC, KP, D), jnp.float32),
          jax.ShapeDtypeStruct((NC, KP), jnp.float32),
      ],
      mesh=mesh,
      compiler_params=cparams,
      scratch_types=(
          [
              pltpu.VMEM((LBLK, D), jnp.float32),
              pltpu.VMEM((LBLK, D), jnp.float32),
              pltpu.VMEM((LBLK, D), jnp.float32),
              pltpu.VMEM((NBLK, BLK), jnp.int32),
              pltpu.VMEM((KP,), jnp.float32),
              pltpu.VMEM((ZR, D), jnp.float32),
              pltpu.VMEM((NS, ZR), jnp.float32),
              pltpu.VMEM((ZR,), jnp.float32),
              pltpu.VMEM_SHARED((KP, D), jnp.float32),
              pltpu.VMEM_SHARED((NS, KP), jnp.float32),
          ]
          + [pltpu.SemaphoreType.DMA] * (2 * NB)
      ),
  )
  def kern(emb_hbm, lab_hbm, sums_hbm, cnts_hbm, rows0_v, rows1_v, rows2_v,
           lab_v, cnt_v, zrow_v, red_v, cout_v, ssums, scnt_s, *sems):
    rows_ring = (rows0_v, rows1_v, rows2_v)
    lsem = sems[:NB]
    ssem = sems[NB:]
    ci = lax.axis_index("c")
    si = lax.axis_index("s")
    wid = ci * NS + si
    base = wid * CHUNK

    zero16 = jnp.zeros((16,), jnp.float32)
    one16 = jnp.full((16,), 1.0, jnp.float32)

    @pl.loop(0, ZR)
    def _(r):
      @pl.loop(0, D, step=16)
      def _(cc):
        zrow_v[r, pl.ds(cc, 16)] = zero16

    @pl.loop(0, KP, step=16)
    def _(r):
      cnt_v[pl.ds(r, 16)] = zero16

    pltpu.sync_copy(zrow_v, ssums.at[pl.ds(si * ZR, ZR)])
    plsc.subcore_barrier()

    pltpu.sync_copy(lab_hbm.at[wid], lab_v)

    def load(b, lblk, nrows):
      st = base + lblk * LBLK
      pltpu.async_copy(emb_hbm.at[pl.ds(st, nrows)],
                       rows_ring[b].at[pl.ds(0, nrows)], lsem[b])

    def wait_load(b, lblk, nrows):
      st = base + lblk * LBLK
      pltpu.make_async_copy(emb_hbm.at[pl.ds(st, nrows)],
                            rows_ring[b].at[pl.ds(0, nrows)],
                            lsem[b]).wait()

    def process(b, lblk, nsub):
      wait_load(b, lblk, nsub * BLK)
      for s in range(nsub):
        pltpu.async_copy(rows_ring[b].at[pl.ds(s * BLK, BLK)],
                         ssums.at[lab_v.at[lblk * SUB + s]], ssem[b],
                         add=True)
      for s in range(nsub):
        for g in range(BLK // 16):
          idx = lab_v[lblk * SUB + s, pl.ds(g * 16, 16)]
          plsc.addupdate_scatter(cnt_v, [idx], one16)

    def drain(b, lblk, nsub):
      for s in range(nsub):
        pltpu.make_async_copy(rows_ring[b].at[pl.ds(s * BLK, BLK)],
                              ssums.at[lab_v.at[lblk * SUB + s]],
                              ssem[b]).wait()

    for b in range(NB):
      load(b, b, LBLK)

    @pl.loop(0, 13)
    def _(o):
      for b in range(NB):
        process(b, o * NB + b, SUB)

      @pl.when(o < 12)
      def _():
        for b in range(NB):
          drain(b, o * NB + b, SUB)
          load(b, (o + 1) * NB + b, LBLK)

      @pl.when(o == 12)
      def _():
        for b in range(NB):
          drain(b, o * NB + b, SUB)
        load(0, 39, LBLK)
        load(1, 40, LBLK)
        load(2, 41, TSUB * BLK)

    process(0, 39, SUB)
    process(1, 40, SUB)
    process(2, 41, TSUB)
    drain(0, 39, SUB)
    drain(1, 40, SUB)
    drain(2, 41, TSUB)

    pltpu.sync_copy(cnt_v, scnt_s.at[si])
    plsc.subcore_barrier()
    for r in range(NS):
      pltpu.sync_copy(scnt_s.at[r, pl.ds(si * ZR, ZR)], red_v.at[r])
    for c in range(0, ZR, 16):
      acc = zero16
      for r in range(NS):
        acc = acc + red_v[r, pl.ds(c, 16)]
      cout_v[pl.ds(c, 16)] = acc
    pltpu.sync_copy(cout_v, cnts_hbm.at[ci, pl.ds(si * ZR, ZR)])

    pltpu.sync_copy(ssums.at[pl.ds(si * ZR, ZR)],
                    sums_hbm.at[ci, pl.ds(si * ZR, ZR)])

  return kern(embeddings, labels)


def _tc_finalize(psums, pcnts):
  def body(s_ref, c_ref, o_ref):
    s = s_ref[0] + s_ref[1]
    cnt = c_ref[0] + c_ref[1]
    pm = (cnt > 0.0).astype(jnp.float32)
    npres = jnp.sum(pm)
    means = s / jnp.maximum(cnt, 1.0)
    overall = jnp.sum(means * pm, axis=0, keepdims=True) / npres
    diff = (means - overall) * pm
    var = jnp.sum(diff * diff, axis=0, keepdims=True) / (npres - 1.0)
    o_ref[...] = jnp.broadcast_to(-jnp.mean(var), (1, 1))

  return pl.pallas_call(
      body,
      out_shape=jax.ShapeDtypeStruct((1, 1), jnp.float32),
  )(psums, pcnts)


def kernel(embeddings, labels):
  labels = labels.astype(jnp.int32).reshape(NW, NBLK, BLK)
  psums, pcnts = _sc_segment_sums(embeddings, labels)
  return _tc_finalize(psums, pcnts.reshape(NC, KP, 1))[0, 0]
